# Initial kernel scaffold; baseline (speedup 1.0000x reference)
import jax
import jax.numpy as jnp
from jax.experimental import pallas as pl


def kernel(x, boxes_per_cls, W, b):
    raise NotImplementedError("write your pallas kernel here")


# speedup vs baseline: 15.5962x; 15.5962x over previous
import jax
import jax.numpy as jnp
from jax.experimental import pallas as pl
from jax.experimental.pallas import tpu as pltpu


def _nms_kernel(x_ref, w_ref, b_ref, bx_ref, logits_ref, preds_ref, dT, idxm):
    C, N = dT.shape
    logits = jax.lax.dot_general(
        x_ref[...], w_ref[...],
        dimension_numbers=(((1,), (1,)), ((), ())),
        preferred_element_type=jnp.float32,
    )
    logits = logits + b_ref[...]
    logits_ref[...] = logits
    probs = jax.nn.softmax(logits, axis=1)
    lane_c = jax.lax.broadcasted_iota(jnp.int32, (1, C), 1)
    probs = jnp.where(lane_c == 0, 0.0, probs)
    dT[...] = probs.T
    idxm[...] = (jax.lax.broadcasted_iota(jnp.int32, (C, N), 1) * C
                 + jax.lax.broadcasted_iota(jnp.int32, (C, N), 0))
    preds_ref[...] = jnp.zeros((1, N), jnp.int32)
    lid = jax.lax.broadcasted_iota(jnp.int32, (1, N), 1)

    def body(i, carry):
        d = dT[...]
        m = jnp.max(d)
        idx = jnp.min(jnp.where(d == m, idxm[...], jnp.int32(2 ** 30)))
        box = idx // C
        cls = idx - box * C
        selm = lid == box
        preds_ref[...] = jnp.where(selm, cls, preds_ref[...])
        sl = bx_ref[pl.ds(cls, 1), :, :][0]
        x1 = sl[0:1]
        y1 = sl[1:2]
        x2 = sl[2:3]
        y2 = sl[3:4]
        sx1 = jnp.sum(jnp.where(selm, x1, 0.0))
        sy1 = jnp.sum(jnp.where(selm, y1, 0.0))
        sx2 = jnp.sum(jnp.where(selm, x2, 0.0))
        sy2 = jnp.sum(jnp.where(selm, y2, 0.0))
        iw = jnp.maximum(jnp.minimum(x2, sx2) - jnp.maximum(x1, sx1) + 1.0, 0.0)
        ih = jnp.maximum(jnp.minimum(y2, sy2) - jnp.maximum(y1, sy1) + 1.0, 0.0)
        inters = iw * ih
        area = (x2 - x1 + 1.0) * (y2 - y1 + 1.0)
        sarea = (sx2 - sx1 + 1.0) * (sy2 - sy1 + 1.0)
        union = area + sarea - inters
        mask = (inters / union) >= 0.5
        row = dT[pl.ds(cls, 1), :]
        dT[pl.ds(cls, 1), :] = jnp.where(mask, 0.0, row)
        d2 = dT[...]
        lane2 = jax.lax.broadcasted_iota(jnp.int32, (C, N), 1)
        dT[...] = jnp.where(lane2 == box, -1.0, d2)
        return carry

    jax.lax.fori_loop(0, N, body, 0)


def kernel(x, boxes_per_cls, W, b):
    N, D = x.shape
    C = W.shape[0]
    boxesT = jnp.transpose(boxes_per_cls, (1, 2, 0))
    b2 = b.reshape(1, C)
    logits, preds = pl.pallas_call(
        _nms_kernel,
        out_shape=(
            jax.ShapeDtypeStruct((N, C), jnp.float32),
            jax.ShapeDtypeStruct((1, N), jnp.int32),
        ),
        scratch_shapes=[
            pltpu.VMEM((C, N), jnp.float32),
            pltpu.VMEM((C, N), jnp.int32),
        ],
    )(x, W, b2, boxesT)
    return logits, preds.reshape(N)
